# baseline (speedup 1.0000x reference)
import functools
import jax
import jax.numpy as jnp
from jax import lax
from jax.experimental import pallas as pl
from jax.experimental.pallas import tpu as pltpu
from jax.experimental.pallas import tpu_sc as plsc

HID = 64
VOCAB = 64
ROWS = 4096
COLS = 200
B = ROWS * COLS
NPAIR = B // 2
PW = 4 * HID
NC = 2
NS = 16
NW = NC * NS
PAIRS_PER_W = NPAIR // NW
UNIT = 128
NUNIT = PAIRS_PER_W // UNIT

NBUF = 3
DEPTH = 1


def _prep_body(x_ref, table_ref, w1_ref, b1_ref, w2_ref, b2_ref, gamma_ref,
               beta_ref, ptab_ref, xp_ref):
    h = table_ref[...]
    z = jnp.dot(h, w1_ref[...], preferred_element_type=jnp.float32) + b1_ref[...]
    z = jnp.maximum(z, 0.0)
    ff = jnp.dot(z, w2_ref[...], preferred_element_type=jnp.float32) + b2_ref[...]
    s = h + ff
    mu = jnp.mean(s, axis=-1, keepdims=True)
    var = jnp.mean((s - mu) * (s - mu), axis=-1, keepdims=True)
    shat = (s - mu) * lax.rsqrt(var + 1e-5)
    t = shat * gamma_ref[...] + beta_ref[...]

    b1t = jnp.broadcast_to(t[:, None, :], (VOCAB, VOCAB, HID))
    b2t = jnp.broadcast_to(t[None, :, :], (VOCAB, VOCAB, HID))
    ptab_ref[...] = jnp.concatenate(
        [b1t.reshape(VOCAB * VOCAB, HID),
         b2t.reshape(VOCAB * VOCAB, HID)], axis=1)

    xf = x_ref[...].astype(jnp.float32)
    rows = lax.broadcasted_iota(jnp.int32, (128, 64), 0)
    cols = lax.broadcasted_iota(jnp.int32, (128, 64), 1)
    sel_even = jnp.where(rows == 2 * cols, 1.0, 0.0).astype(jnp.float32)
    sel_odd = jnp.where(rows == 2 * cols + 1, 1.0, 0.0).astype(jnp.float32)
    xe = jnp.dot(xf, sel_even, preferred_element_type=jnp.float32)
    xo = jnp.dot(xf, sel_odd, preferred_element_type=jnp.float32)
    xp_ref[...] = (xe * 64.0 + xo).astype(jnp.int32)


def _prepare(x4, table, w1, b1, w2, b2, gamma, beta):
    return pl.pallas_call(
        _prep_body,
        out_shape=(
            jax.ShapeDtypeStruct((VOCAB * VOCAB, 2 * HID), jnp.float32),
            jax.ShapeDtypeStruct((B // 128, 64), jnp.int32),
        ),
    )(x4, table, w1, b1.reshape(1, -1), w2, b2.reshape(1, -1),
      gamma.reshape(1, -1), beta.reshape(1, -1))


@functools.cache
def _make_gather():
    mesh = plsc.VectorSubcoreMesh(core_axis_name="c", subcore_axis_name="s")

    @functools.partial(
        pl.kernel,
        out_type=jax.ShapeDtypeStruct((NPAIR // 4, 4 * PW), jnp.float32),
        mesh=mesh,
        scratch_types=[
            pltpu.VMEM((PAIRS_PER_W,), jnp.int32),
            pltpu.VMEM((NBUF, UNIT, 2 * HID), jnp.float32),
            pltpu.SemaphoreType.DMA,
            pltpu.SemaphoreType.DMA,
        ],
        compiler_params=pltpu.CompilerParams(use_tc_tiling_on_sc=False),
    )
    def _gather(xp_hbm, ptab_hbm, out_hbm, idx_v, rows_v, gsem, ssem):
        wid = lax.axis_index("s") * NC + lax.axis_index("c")
        pltpu.sync_copy(xp_hbm.at[wid], idx_v)
        pair0 = wid * PAIRS_PER_W

        def gather_copy(u, b):
            return pltpu.make_async_copy(
                ptab_hbm.at[idx_v.at[pl.ds(u * UNIT, UNIT)]],
                rows_v.at[b], gsem)

        def scatter_copies(u, b):
            dst = out_hbm.at[pl.ds((pair0 + u * UNIT) // 4, UNIT // 4)]
            copies = []
            for k in range(4):
                src = rows_v.at[b, pl.ds(32 * k, 32)]
                copies.append(pltpu.make_async_copy(
                    src.at[slice(None), pl.ds(0, HID)],
                    dst.at[slice(None), pl.ds(k * PW, HID)], ssem))
                copies.append(pltpu.make_async_copy(
                    src.at[slice(None), pl.ds(HID, HID)],
                    dst.at[slice(None), pl.ds(k * PW + 2 * HID, HID)], ssem))
            return copies

        def body(j, carry):
            @pl.when(j < NUNIT)
            def _():
                b = j % NBUF

                @pl.when(j >= NBUF)
                def _():
                    for c in scatter_copies(j - NBUF, b):
                        c.wait()

                gather_copy(j, b).start()

            @pl.when(j >= DEPTH)
            def _():
                i = j - DEPTH
                bi = i % NBUF
                gather_copy(i, bi).wait()
                for c in scatter_copies(i, bi):
                    c.start()

            return carry

        lax.fori_loop(0, NUNIT + DEPTH, body, 0)

        def drain(j, carry):
            for c in scatter_copies(j, j % NBUF):
                c.wait()
            return carry

        lax.fori_loop(NUNIT - NBUF, NUNIT, drain, 0)

    return _gather


def kernel(x, table, W1, b1, W2, b2, gamma, beta):
    x4 = x.reshape(B // 128, 128).astype(jnp.int32)
    ptab, xp = _prepare(x4, table, W1, b1, W2, b2, gamma, beta)
    xp_w = (xp.reshape(NW, NUNIT, 32, 4)
            .transpose(0, 1, 3, 2)
            .reshape(NW, PAIRS_PER_W))
    out = _make_gather()(xp_w, ptab)
    return out.reshape(B, 2 * HID)[:, :HID].reshape(ROWS, COLS, HID)


# speedup vs baseline: 1.1761x; 1.1761x over previous
import functools
import jax
import jax.numpy as jnp
from jax import lax
from jax.experimental import pallas as pl
from jax.experimental.pallas import tpu as pltpu
from jax.experimental.pallas import tpu_sc as plsc

HID = 64
VOCAB = 64
ROWS = 4096
COLS = 200
B = ROWS * COLS
NPAIR = B // 2
PW = 4 * HID
NC = 2
NS = 16
NW = NC * NS
PAIRS_PER_W = NPAIR // NW
UNIT = 128
NUNIT = PAIRS_PER_W // UNIT

NBUF = 3
DEPTH = 1


def _prep_body(x_ref, table_ref, w1_ref, b1_ref, w2_ref, b2_ref, gamma_ref,
               beta_ref, ptab_ref, xp_ref):
    h = table_ref[...]
    z = jnp.dot(h, w1_ref[...], preferred_element_type=jnp.float32) + b1_ref[...]
    z = jnp.maximum(z, 0.0)
    ff = jnp.dot(z, w2_ref[...], preferred_element_type=jnp.float32) + b2_ref[...]
    s = h + ff
    mu = jnp.mean(s, axis=-1, keepdims=True)
    var = jnp.mean((s - mu) * (s - mu), axis=-1, keepdims=True)
    shat = (s - mu) * lax.rsqrt(var + 1e-5)
    t = shat * gamma_ref[...] + beta_ref[...]

    b1t = jnp.broadcast_to(t[:, None, :], (VOCAB, VOCAB, HID))
    b2t = jnp.broadcast_to(t[None, :, :], (VOCAB, VOCAB, HID))
    ptab_ref[...] = jnp.concatenate(
        [b1t.reshape(VOCAB * VOCAB, HID),
         b2t.reshape(VOCAB * VOCAB, HID)], axis=1)

    xf = x_ref[...].astype(jnp.float32)
    rows = lax.broadcasted_iota(jnp.int32, (128, 64), 0)
    cols = lax.broadcasted_iota(jnp.int32, (128, 64), 1)
    sel_even = jnp.where(rows == 2 * cols, 1.0, 0.0).astype(jnp.float32)
    sel_odd = jnp.where(rows == 2 * cols + 1, 1.0, 0.0).astype(jnp.float32)
    xe = jnp.dot(xf, sel_even, preferred_element_type=jnp.float32)
    xo = jnp.dot(xf, sel_odd, preferred_element_type=jnp.float32)
    xp_ref[...] = (xe * 64.0 + xo).astype(jnp.int32)


def _prepare(x4, table, w1, b1, w2, b2, gamma, beta):
    return pl.pallas_call(
        _prep_body,
        out_shape=(
            jax.ShapeDtypeStruct((VOCAB * VOCAB, 2 * HID), jnp.float32),
            jax.ShapeDtypeStruct((B // 128, 64), jnp.int32),
        ),
    )(x4, table, w1, b1.reshape(1, -1), w2, b2.reshape(1, -1),
      gamma.reshape(1, -1), beta.reshape(1, -1))


@functools.cache
def _make_gather():
    mesh = plsc.VectorSubcoreMesh(core_axis_name="c", subcore_axis_name="s")

    @functools.partial(
        pl.kernel,
        out_type=jax.ShapeDtypeStruct((NPAIR, PW), jnp.float32),
        mesh=mesh,
        scratch_types=[
            pltpu.VMEM((PAIRS_PER_W,), jnp.int32),
            pltpu.VMEM((NBUF, UNIT, 2 * HID), jnp.float32),
            pltpu.SemaphoreType.DMA,
            pltpu.SemaphoreType.DMA,
        ],
        compiler_params=pltpu.CompilerParams(use_tc_tiling_on_sc=False),
    )
    def _gather(xp_hbm, ptab_hbm, out_hbm, idx_v, rows_v, gsem, ssem):
        wid = lax.axis_index("s") * NC + lax.axis_index("c")
        pltpu.sync_copy(xp_hbm.at[wid], idx_v)
        pair0 = wid * PAIRS_PER_W

        def gather_copy(u, b):
            return pltpu.make_async_copy(
                ptab_hbm.at[idx_v.at[pl.ds(u * UNIT, UNIT)]],
                rows_v.at[b], gsem)

        def scatter_copies(u, b):
            dst = out_hbm.at[pl.ds(pair0 + u * UNIT, UNIT)]
            return (
                pltpu.make_async_copy(
                    rows_v.at[b, slice(None), pl.ds(0, HID)],
                    dst.at[slice(None), pl.ds(0, HID)], ssem),
                pltpu.make_async_copy(
                    rows_v.at[b, slice(None), pl.ds(HID, HID)],
                    dst.at[slice(None), pl.ds(2 * HID, HID)], ssem),
            )

        def body(j, carry):
            @pl.when(j < NUNIT)
            def _():
                b = j % NBUF

                @pl.when(j >= NBUF)
                def _():
                    for c in scatter_copies(j - NBUF, b):
                        c.wait()

                gather_copy(j, b).start()

            @pl.when(j >= DEPTH)
            def _():
                i = j - DEPTH
                bi = i % NBUF
                gather_copy(i, bi).wait()
                for c in scatter_copies(i, bi):
                    c.start()

            return carry

        lax.fori_loop(0, NUNIT + DEPTH, body, 0)

        def drain(j, carry):
            for c in scatter_copies(j, j % NBUF):
                c.wait()
            return carry

        lax.fori_loop(NUNIT - NBUF, NUNIT, drain, 0)

    return _gather


def kernel(x, table, W1, b1, W2, b2, gamma, beta):
    x4 = x.reshape(B // 128, 128).astype(jnp.int32)
    ptab, xp = _prepare(x4, table, W1, b1, W2, b2, gamma, beta)
    xp_w = xp.reshape(NW, PAIRS_PER_W)
    out = _make_gather()(xp_w, ptab)
    return out.reshape(B, 2 * HID)[:, :HID].reshape(ROWS, COLS, HID)
